# baseline (speedup 1.0000x reference)
import functools

import jax
import jax.numpy as jnp
from jax import lax
from jax.experimental import pallas as pl
from jax.experimental.pallas import tpu as pltpu
from jax.experimental.pallas import tpu_sc as plsc

B, N, D, R, K, ORDER, H = 2, 96, 64, 32, 8, 3, 64
BI = 16
NW = 32
KP = 16
NEG = -1e30


def _silu(x):
    return x * jax.nn.sigmoid(x)


def _sc_topk(md, mk, rxf, ryf, rzf):
    apw = (B * N) // NW
    nc = N // 16
    mesh = plsc.VectorSubcoreMesh(core_axis_name="c", subcore_axis_name="s")
    f32 = jnp.float32
    i32 = jnp.int32

    @functools.partial(
        pl.kernel, mesh=mesh,
        out_type=[jax.ShapeDtypeStruct((B * N * KP,), i32)]
        + [jax.ShapeDtypeStruct((B * N * KP,), f32)] * 4,
        scratch_types=[pltpu.VMEM((N,), f32)] * 5
        + [pltpu.VMEM((KP,), i32)] + [pltpu.VMEM((KP,), f32)] * 4,
        compiler_params=pltpu.CompilerParams(needs_layout_passes=False),
    )
    def k(md_h, mk_h, rx_h, ry_h, rz_h, oi_h, om_h, ox_h, oy_h, oz_h,
          md_v, mk_v, rx_v, ry_v, rz_v, oi_v, om_v, ox_v, oy_v, oz_v):
        wid = lax.axis_index("s") * 2 + lax.axis_index("c")
        iota = lax.iota(i32, 16)

        def body(t, carry):
            a = wid * apw + t
            roff = a * N
            pltpu.sync_copy(md_h.at[pl.ds(roff, N)], md_v)
            pltpu.sync_copy(mk_h.at[pl.ds(roff, N)], mk_v)
            pltpu.sync_copy(rx_h.at[pl.ds(roff, N)], rx_v)
            pltpu.sync_copy(ry_h.at[pl.ds(roff, N)], ry_v)
            pltpu.sync_copy(rz_h.at[pl.ds(roff, N)], rz_v)
            v = [md_v[pl.ds(c * 16, 16)] for c in range(nc)]
            oidx = jnp.zeros((16,), i32)
            for r in range(K):
                m = v[0]
                for c in range(1, nc):
                    m = jnp.minimum(m, v[c])
                s = jnp.min(m)
                best = jnp.full((16,), 32767, i32)
                for c in range(nc):
                    eq = v[c] == s
                    pop = plsc.all_reduce_population_count(eq)
                    ffs = plsc.all_reduce_ffs(eq)
                    cand = jnp.where(pop > 0, c * 16 + ffs, 32767)
                    best = jnp.minimum(best, cand)
                oidx = jnp.where(iota == r, best, oidx)
                for c in range(nc):
                    v[c] = jnp.where(c * 16 + iota == best, 1e9, v[c])
            oi_v[...] = oidx
            om_v[...] = plsc.load_gather(mk_v, [oidx])
            ox_v[...] = plsc.load_gather(rx_v, [oidx])
            oy_v[...] = plsc.load_gather(ry_v, [oidx])
            oz_v[...] = plsc.load_gather(rz_v, [oidx])
            ooff = a * KP
            pltpu.sync_copy(oi_v, oi_h.at[pl.ds(ooff, KP)])
            pltpu.sync_copy(om_v, om_h.at[pl.ds(ooff, KP)])
            pltpu.sync_copy(ox_v, ox_h.at[pl.ds(ooff, KP)])
            pltpu.sync_copy(oy_v, oy_h.at[pl.ds(ooff, KP)])
            pltpu.sync_copy(oz_v, oz_h.at[pl.ds(ooff, KP)])
            return carry

        lax.fori_loop(0, apw, body, 0)

    flat = lax.optimization_barrier(
        (md.reshape(-1), mk.reshape(-1),
         rxf.reshape(-1), ryf.reshape(-1), rzf.reshape(-1)))
    oi, om, ox, oy, oz = k(*flat)
    rs = (B, N, KP)
    return (oi.reshape(rs), om.reshape(rs), ox.reshape(rs),
            oy.reshape(rs), oz.reshape(rs))


def _fused_kernel(node_ref, mask_ref, rbf_ref, rx_ref, ry_ref, rz_ref,
                  oi_ref, om_ref, ox_ref, oy_ref, oz_ref,
                  ep_w1, ep_b1, ep_w2, ep_b2,
                  tp_w1, tp_c, tp_b1, tp_w2, tp_b2,
                  ts_w1, ts_b1, ts_w2t, ts_b2,
                  tm_g, tm_b, tm_w1, tm_b1, tm_w2, tm_b2,
                  eg_w, eg_b, no_g, no_b, no_w, no_b2, en_g, en_b,
                  nd_out, es_out):
    f32 = jnp.float32
    i_blk = pl.program_id(1)
    node_b = node_ref[0]
    maskf = mask_ref[0]
    rbf = rbf_ref[0]
    rx = rx_ref[0]
    ry = ry_ref[0]
    rz = rz_ref[0]

    idx = oi_ref[0][:, :K]
    tmask = om_ref[0][:, :K]
    crx = ox_ref[0][:, :K]
    cry = oy_ref[0][:, :K]
    crz = oz_ref[0][:, :K]

    iota_kn = jax.lax.broadcasted_iota(jnp.int32, (BI, K, N), 2)
    onehot = (idx[:, :, None] == iota_kn).astype(f32)

    onehot2 = onehot.reshape(BI * K, N)
    nodecat = jnp.concatenate([node_b, node_b @ ts_w1[D:, :]], axis=1)
    g = onehot2 @ nodecat
    comp_feat = g[:, :D].reshape(BI, K, D)
    cfW = g[:, D:].reshape(BI, K, H)

    rbfW2 = rbf.reshape(BI * N, R) @ tp_w1[ORDER + 1:, :]
    row_a = jax.lax.broadcasted_iota(jnp.int32, (BI, K), 0)
    flatidx = row_a * N + idx
    iota_f = jax.lax.broadcasted_iota(jnp.int32, (BI, K, BI * N), 2)
    onehot_f = (flatidx[:, :, None] == iota_f).astype(f32).reshape(BI * K, BI * N)
    radial_h = (onehot_f @ rbfW2).reshape(BI, K, H)

    cos = (crx[:, :, None] * rx[:, None, :] +
           cry[:, :, None] * ry[:, None, :] +
           crz[:, :, None] * rz[:, None, :])
    cos = jnp.clip(cos, -1.0 + 1e-6, 1.0 - 1e-6)

    def _r4(v):
        return v.reshape(1, 1, 1, -1)

    base = radial_h + tp_c[0:1, :][None, :, :] + tp_b1[...][None, :, :]
    x = cos[..., None]
    th = (x * _r4(tp_c[3:4, :]) + _r4(tp_c[2:3, :])) * x
    th = (th + _r4(tp_c[1:2, :])) * x + base[:, :, None, :]
    th = _silu(th)
    tw = (th.reshape(BI * K * N, H) @ tp_w2[...] + tp_b2[...]).reshape(BI, K, N, D)

    sh = _silu((tw.reshape(BI * K * N, D) @ ts_w1[:D, :]).reshape(BI, K, N, H)
               + cfW[:, :, None, :] + _r4(ts_b1[...]))
    logits = jnp.sum(sh * _r4(ts_w2t[...]), axis=3) + ts_b2[...].reshape(1, 1, 1)

    pm = maskf[:, None, :] * tmask[:, :, None]
    pm = jnp.where(idx[:, :, None] == iota_kn, 0.0, pm)

    logits = jnp.where(pm <= 0.0, NEG, logits)
    lmax = jnp.max(logits, axis=1, keepdims=True)
    e = jnp.exp(logits - lmax)
    attn = e / jnp.sum(e, axis=1, keepdims=True)
    attn = jnp.where(pm > 0.0, attn, 0.0)

    tp_pair = tw * comp_feat[:, :, None, :]
    t_attn = jnp.sum(tp_pair * attn[..., None], axis=1)
    mp = jnp.where(pm[..., None] <= 0.0, NEG, tp_pair)
    t_max = jnp.max(mp, axis=1)
    t_max = jnp.where(t_max <= NEG * 0.5, 0.0, t_max)

    mi = jnp.concatenate([t_attn, t_max], axis=2)
    mu = jnp.mean(mi, axis=2, keepdims=True)
    mv = jnp.mean((mi - mu) ** 2, axis=2, keepdims=True)
    mi = (mi - mu) * (1.0 / jnp.sqrt(mv + 1e-5)) * tm_g[...].reshape(1, 1, 2 * D) \
        + tm_b[...].reshape(1, 1, 2 * D)
    mh = _silu(mi.reshape(BI * N, 2 * D) @ tm_w1[...] + tm_b1[...])
    ctx = mh @ tm_w2[...] + tm_b2[...]

    node_i = node_ref[0, pl.ds(i_blk * BI, BI), :]
    hi = node_i @ ep_w1[:D, :]
    dstW = node_b @ ep_w1[D:2 * D, :]
    rbfW1 = (rbf.reshape(BI * N, R) @ ep_w1[2 * D:, :]).reshape(BI, N, H)
    eh = _silu(hi[:, None, :] + dstW[None, :, :] + rbfW1
               + ep_b1[...].reshape(1, 1, H))
    eb = (eh.reshape(BI * N, H) @ ep_w2[...] + ep_b2[...]).reshape(BI, N, D)
    mask3 = maskf[:, :, None]
    eb = eb * mask3

    ef = eb + ctx.reshape(BI, N, D)
    emu = jnp.mean(ef, axis=2, keepdims=True)
    ev = jnp.mean((ef - emu) ** 2, axis=2, keepdims=True)
    ef = (ef - emu) * (1.0 / jnp.sqrt(ev + 1e-5)) * en_g[...].reshape(1, 1, D) \
        + en_b[...].reshape(1, 1, D)
    gate = jax.nn.sigmoid(ef.reshape(BI * N, D) @ eg_w[...]
                          + eg_b[...]).reshape(BI, N, D)
    ef = gate * ef

    ns = jnp.sum(ef * mask3, axis=1)
    es = jnp.sum(ef, axis=1)

    nmu = jnp.mean(ns, axis=1, keepdims=True)
    nv = jnp.mean((ns - nmu) ** 2, axis=1, keepdims=True)
    nd = (ns - nmu) * (1.0 / jnp.sqrt(nv + 1e-5)) * no_g[...] + no_b[...]
    nd = nd @ no_w[...] + no_b2[...]

    nd_out[0] = nd
    es_out[0] = es


def kernel(node_s, dist, rbf, r_hat, access_mask, params):
    p = params
    f32 = jnp.float32
    maskf = access_mask.astype(f32)
    maxd = jnp.maximum(dist.max(axis=(1, 2), keepdims=True), 1.0) + 1.0
    md = jnp.where(access_mask, dist, maxd)
    rx = r_hat[..., 0]
    ry = r_hat[..., 1]
    rz = r_hat[..., 2]

    oi, om, ox, oy, oz = _sc_topk(md, maskf, rx, ry, rz)

    def row2(v):
        return v.reshape(1, -1)

    w = p['tp_w1']
    tp_c = jnp.stack([w[0] - 0.5 * w[2], w[1] - 1.5 * w[3],
                      1.5 * w[2], 2.5 * w[3]], axis=0)

    args = (node_s, maskf, rbf, rx, ry, rz, oi, om, ox, oy, oz,
            p['ep_w1'], row2(p['ep_b1']), p['ep_w2'], row2(p['ep_b2']),
            p['tp_w1'], tp_c, row2(p['tp_b1']), p['tp_w2'], row2(p['tp_b2']),
            p['ts_w1'], row2(p['ts_b1']), p['ts_w2'].T, row2(p['ts_b2']),
            row2(p['tm_g']), row2(p['tm_b']),
            p['tm_w1'], row2(p['tm_b1']), p['tm_w2'], row2(p['tm_b2']),
            p['eg_w'], row2(p['eg_b']), row2(p['no_g']), row2(p['no_b']),
            p['no_w'], row2(p['no_b2']), row2(p['en_g']), row2(p['en_b']))

    def full(a):
        return pl.BlockSpec(a.shape, lambda b, i: (0,) * a.ndim)

    row_specs = [
        pl.BlockSpec((1, N, D), lambda b, i: (b, 0, 0)),
        pl.BlockSpec((1, BI, N), lambda b, i: (b, i, 0)),
        pl.BlockSpec((1, BI, N, R), lambda b, i: (b, i, 0, 0)),
        pl.BlockSpec((1, BI, N), lambda b, i: (b, i, 0)),
        pl.BlockSpec((1, BI, N), lambda b, i: (b, i, 0)),
        pl.BlockSpec((1, BI, N), lambda b, i: (b, i, 0)),
        pl.BlockSpec((1, BI, KP), lambda b, i: (b, i, 0)),
        pl.BlockSpec((1, BI, KP), lambda b, i: (b, i, 0)),
        pl.BlockSpec((1, BI, KP), lambda b, i: (b, i, 0)),
        pl.BlockSpec((1, BI, KP), lambda b, i: (b, i, 0)),
        pl.BlockSpec((1, BI, KP), lambda b, i: (b, i, 0)),
    ]
    in_specs = row_specs + [full(a) for a in args[11:]]

    nd, es = pl.pallas_call(
        _fused_kernel,
        grid=(B, N // BI),
        in_specs=in_specs,
        out_specs=[
            pl.BlockSpec((1, BI, D), lambda b, i: (b, i, 0)),
            pl.BlockSpec((1, BI, D), lambda b, i: (b, i, 0)),
        ],
        out_shape=[
            jax.ShapeDtypeStruct((B, N, D), f32),
            jax.ShapeDtypeStruct((B, N, D), f32),
        ],
    )(*args)

    denom = jnp.maximum(maskf.sum(axis=(1, 2)), 1.0)[:, None]
    bond_graph = es.sum(axis=1) / denom
    return nd, bond_graph
